# baseline (speedup 1.0000x reference)
import functools

import jax
import jax.numpy as jnp
from jax import lax
from jax.experimental import pallas as pl
from jax.experimental.pallas import tpu as pltpu
from jax.experimental.pallas import tpu_sc as plsc

_N = 4096
_D = 1024
_E = 8
_F = 2048
_TM = 256
_NB = _N // _TM
_NT = 23
_NPAD = _NT * _TM
_FB = 512
_NF = _F // _FB
_NW = 32
_RPW = _N // _NW
_CH = 64
_NCH = _RPW // _CH


def _router_body(x_ref, rw_ref, mask_ref, xg_ref, e_ref, rank_ref, cnt_ref, acc):
    i = pl.program_id(0)

    @pl.when(i == 0)
    def _():
        acc[...] = jnp.zeros_like(acc)

    xb = x_ref[...]
    logits = lax.dot_general(xb, rw_ref[...], (((1,), (1,)), ((), ())),
                             preferred_element_type=jnp.float32)
    m = jnp.max(logits, axis=-1, keepdims=True)
    ex = jnp.exp(logits - m)
    s = jnp.sum(ex, axis=-1, keepdims=True)
    probs = ex / s
    pm = jnp.max(probs, axis=-1, keepdims=True)
    lane = lax.broadcasted_iota(jnp.int32, (_TM, _E), 1).astype(jnp.float32)
    cand = jnp.where(probs >= pm, lane, float(_E))
    e = jnp.min(cand, axis=-1, keepdims=True)
    oh = (lane == e).astype(jnp.float32)
    r0 = lax.broadcasted_iota(jnp.int32, (_TM, _TM), 0)
    c0 = lax.broadcasted_iota(jnp.int32, (_TM, _TM), 1)
    tri = (c0 < r0).astype(jnp.float32)
    rank_all = jnp.dot(tri, oh, preferred_element_type=jnp.float32)
    base = acc[...]
    rank_tok = jnp.sum((rank_all + base) * oh, axis=-1, keepdims=True)
    gate = pm * mask_ref[...]
    xg_ref[...] = xb * gate
    e_ref[...] = e
    rank_ref[...] = rank_tok
    acc[...] = base + jnp.sum(oh, axis=0, keepdims=True)

    @pl.when(i == _NB - 1)
    def _():
        cnt_ref[...] = acc[...]


def _plan_body(cnt_ref, e_ref, rank_ref, p_ref, te_ref, act_ref, offs_s):
    i = pl.program_id(0)

    @pl.when(i == 0)
    def _():
        c = cnt_ref[...]
        a = jnp.ceil(c * (1.0 / _TM)) * float(_TM)
        r0 = lax.broadcasted_iota(jnp.int32, (_E, _E), 0)
        c0 = lax.broadcasted_iota(jnp.int32, (_E, _E), 1)
        triu = (r0 < c0).astype(jnp.float32)
        offs = jnp.dot(a, triu, preferred_element_type=jnp.float32)
        offs_s[...] = offs
        ends = offs + a
        eye = (r0 == c0).astype(jnp.float32)
        ends_col = lax.dot_general(eye, ends, (((1,), (1,)), ((), ())),
                                   preferred_element_type=jnp.float32)
        tt = lax.broadcasted_iota(jnp.int32, (_E, 128), 1).astype(jnp.float32) * float(_TM)
        te = jnp.sum((ends_col <= tt).astype(jnp.float32), axis=0, keepdims=True)
        te_ref[...] = jnp.minimum(te, float(_E - 1)).astype(jnp.int32)
        total = jnp.sum(a, axis=-1, keepdims=True)
        trow = lax.broadcasted_iota(jnp.int32, (1, 128), 1).astype(jnp.float32) * float(_TM)
        act_ref[...] = (trow < total).astype(jnp.int32)

    e = e_ref[...]
    lane = lax.broadcasted_iota(jnp.int32, (_TM, _E), 1).astype(jnp.float32)
    oh = (lane == e).astype(jnp.float32)
    off_tok = jnp.sum(oh * offs_s[...], axis=-1, keepdims=True)
    p_ref[...] = (off_tok + rank_ref[...]).astype(jnp.int32)


def _ffn_body(te_ref, act_ref, xs_ref, w1_ref, w2_ref, o_ref, acc_ref):
    f = pl.program_id(1)

    @pl.when(act_ref[pl.program_id(0)] == 1)
    def _():
        xb16 = xs_ref[...].astype(jnp.bfloat16)
        h = jnp.dot(xb16, w1_ref[0], preferred_element_type=jnp.float32)
        h = jnp.maximum(h, 0.0).astype(jnp.bfloat16)
        prod = jnp.dot(h, w2_ref[0], preferred_element_type=jnp.float32)

        @pl.when(f == 0)
        def _():
            acc_ref[...] = prod

        @pl.when(f > 0)
        def _():
            acc_ref[...] = acc_ref[...] + prod

        @pl.when(f == _NF - 1)
        def _():
            o_ref[...] = acc_ref[...]


@functools.cache
def _sc_kernels():
    mesh = plsc.VectorSubcoreMesh(core_axis_name="c", subcore_axis_name="s")
    scratch = [
        pltpu.VMEM((_NCH, _CH), jnp.int32),
        pltpu.VMEM((_CH, _D), jnp.float32),
        pltpu.SemaphoreType.DMA,
    ]

    @functools.partial(
        pl.kernel,
        out_type=jax.ShapeDtypeStruct((_NPAD, _D), jnp.float32),
        mesh=mesh,
        scratch_types=scratch,
    )
    def sc_scatter(xg_hbm, p_hbm, xs_hbm, idx_v, rows_v, sem):
        wid = lax.axis_index("s") * 2 + lax.axis_index("c")
        base = wid * _RPW
        pltpu.sync_copy(p_hbm.at[wid], idx_v)
        for c in range(_NCH):
            pltpu.sync_copy(xg_hbm.at[pl.ds(base + c * _CH, _CH)], rows_v)
            pltpu.async_copy(rows_v, xs_hbm.at[idx_v.at[c]], sem).wait()

    @functools.partial(
        pl.kernel,
        out_type=jax.ShapeDtypeStruct((_N, _D), jnp.float32),
        mesh=mesh,
        scratch_types=scratch,
    )
    def sc_gather(os_hbm, p_hbm, y_hbm, idx_v, rows_v, sem):
        wid = lax.axis_index("s") * 2 + lax.axis_index("c")
        base = wid * _RPW
        pltpu.sync_copy(p_hbm.at[wid], idx_v)
        for c in range(_NCH):
            pltpu.async_copy(os_hbm.at[idx_v.at[c]], rows_v, sem).wait()
            pltpu.sync_copy(rows_v, y_hbm.at[pl.ds(base + c * _CH, _CH)])

    return sc_scatter, sc_gather


def kernel(x, token_mask, router_w, w1, w2):
    Bc, Tc, Dc = x.shape
    xf = x.reshape(_N, _D)
    maskf = token_mask.reshape(_N, 1).astype(jnp.float32)

    router_call = pl.pallas_call(
        _router_body,
        grid=(_NB,),
        in_specs=[
            pl.BlockSpec((_TM, _D), lambda i: (i, 0)),
            pl.BlockSpec((_E, _D), lambda i: (0, 0)),
            pl.BlockSpec((_TM, 1), lambda i: (i, 0)),
        ],
        out_specs=[
            pl.BlockSpec((_TM, _D), lambda i: (i, 0)),
            pl.BlockSpec((_TM, 1), lambda i: (i, 0)),
            pl.BlockSpec((_TM, 1), lambda i: (i, 0)),
            pl.BlockSpec((1, _E), lambda i: (0, 0)),
        ],
        out_shape=[
            jax.ShapeDtypeStruct((_N, _D), jnp.float32),
            jax.ShapeDtypeStruct((_N, 1), jnp.float32),
            jax.ShapeDtypeStruct((_N, 1), jnp.float32),
            jax.ShapeDtypeStruct((1, _E), jnp.float32),
        ],
        scratch_shapes=[pltpu.VMEM((1, _E), jnp.float32)],
    )
    xg, ef, rank, counts = router_call(xf, router_w, maskf)

    plan_call = pl.pallas_call(
        _plan_body,
        grid=(_NB,),
        in_specs=[
            pl.BlockSpec((1, _E), lambda i: (0, 0)),
            pl.BlockSpec((_TM, 1), lambda i: (i, 0)),
            pl.BlockSpec((_TM, 1), lambda i: (i, 0)),
        ],
        out_specs=[
            pl.BlockSpec((_TM, 1), lambda i: (i, 0)),
            pl.BlockSpec((1, 128), lambda i: (0, 0)),
            pl.BlockSpec((1, 128), lambda i: (0, 0)),
        ],
        out_shape=[
            jax.ShapeDtypeStruct((_N, 1), jnp.int32),
            jax.ShapeDtypeStruct((1, 128), jnp.int32),
            jax.ShapeDtypeStruct((1, 128), jnp.int32),
        ],
        scratch_shapes=[pltpu.VMEM((1, _E), jnp.float32)],
    )
    p, te_pad, act_pad = plan_call(counts, ef, rank)

    p3 = p.reshape(_NW, _NCH, _CH)
    te = te_pad.reshape(128)[:_NT]
    act = act_pad.reshape(128)[:_NT]

    sc_scatter, sc_gather = _sc_kernels()
    xs = sc_scatter(xg, p3)

    ffn_call = pl.pallas_call(
        _ffn_body,
        grid_spec=pltpu.PrefetchScalarGridSpec(
            num_scalar_prefetch=2,
            grid=(_NT, _NF),
            in_specs=[
                pl.BlockSpec((_TM, _D), lambda t, f, te_r, act_r: (t, 0)),
                pl.BlockSpec((1, _D, _FB), lambda t, f, te_r, act_r: (te_r[t], 0, f)),
                pl.BlockSpec((1, _FB, _D), lambda t, f, te_r, act_r: (te_r[t], f, 0)),
            ],
            out_specs=pl.BlockSpec((_TM, _D), lambda t, f, te_r, act_r: (t, 0)),
            scratch_shapes=[pltpu.VMEM((_TM, _D), jnp.float32)],
        ),
        out_shape=jax.ShapeDtypeStruct((_NPAD, _D), jnp.float32),
        compiler_params=pltpu.CompilerParams(
            dimension_semantics=("arbitrary", "arbitrary")),
    )
    osrt = ffn_call(te, act, xs, w1.astype(jnp.bfloat16), w2.astype(jnp.bfloat16))

    y = sc_gather(osrt, p3)
    return y.reshape(Bc, Tc, Dc)


# speedup vs baseline: 1.5131x; 1.5131x over previous
import functools

import jax
import jax.numpy as jnp
from jax import lax
from jax.experimental import pallas as pl
from jax.experimental.pallas import tpu as pltpu
from jax.experimental.pallas import tpu_sc as plsc

_N = 4096
_D = 1024
_E = 8
_F = 2048
_TM = 256
_NB = _N // _TM
_NT = 23
_NPAD = _NT * _TM
_FB = 512
_NF = _F // _FB
_NW = 32
_RPW = _N // _NW
_CH = 64
_NCH = _RPW // _CH


def _router_body(x_ref, rw_ref, mask_ref, xg_ref, e_ref, rank_ref, cnt_ref, acc):
    i = pl.program_id(0)

    @pl.when(i == 0)
    def _():
        acc[...] = jnp.zeros_like(acc)

    xb = x_ref[...]
    logits = lax.dot_general(xb, rw_ref[...], (((1,), (1,)), ((), ())),
                             preferred_element_type=jnp.float32)
    m = jnp.max(logits, axis=-1, keepdims=True)
    ex = jnp.exp(logits - m)
    s = jnp.sum(ex, axis=-1, keepdims=True)
    probs = ex / s
    pm = jnp.max(probs, axis=-1, keepdims=True)
    lane = lax.broadcasted_iota(jnp.int32, (_TM, _E), 1).astype(jnp.float32)
    cand = jnp.where(probs >= pm, lane, float(_E))
    e = jnp.min(cand, axis=-1, keepdims=True)
    oh = (lane == e).astype(jnp.float32)
    r0 = lax.broadcasted_iota(jnp.int32, (_TM, _TM), 0)
    c0 = lax.broadcasted_iota(jnp.int32, (_TM, _TM), 1)
    tri = (c0 < r0).astype(jnp.float32)
    rank_all = jnp.dot(tri, oh, preferred_element_type=jnp.float32)
    base = acc[...]
    rank_tok = jnp.sum((rank_all + base) * oh, axis=-1, keepdims=True)
    gate = pm * mask_ref[...]
    xg_ref[...] = xb * gate
    e_ref[...] = e
    rank_ref[...] = rank_tok
    acc[...] = base + jnp.sum(oh, axis=0, keepdims=True)

    @pl.when(i == _NB - 1)
    def _():
        cnt_ref[...] = acc[...]


def _plan_body(cnt_ref, e_ref, rank_ref, p_ref, te_ref, act_ref, offs_s):
    i = pl.program_id(0)

    @pl.when(i == 0)
    def _():
        c = cnt_ref[...]
        a = jnp.ceil(c * (1.0 / _TM)) * float(_TM)
        r0 = lax.broadcasted_iota(jnp.int32, (_E, _E), 0)
        c0 = lax.broadcasted_iota(jnp.int32, (_E, _E), 1)
        triu = (r0 < c0).astype(jnp.float32)
        offs = jnp.dot(a, triu, preferred_element_type=jnp.float32)
        offs_s[...] = offs
        ends = offs + a
        eye = (r0 == c0).astype(jnp.float32)
        ends_col = lax.dot_general(eye, ends, (((1,), (1,)), ((), ())),
                                   preferred_element_type=jnp.float32)
        tt = lax.broadcasted_iota(jnp.int32, (_E, 128), 1).astype(jnp.float32) * float(_TM)
        te = jnp.sum((ends_col <= tt).astype(jnp.float32), axis=0, keepdims=True)
        te_ref[...] = jnp.minimum(te, float(_E - 1)).astype(jnp.int32)
        total = jnp.sum(a, axis=-1, keepdims=True)
        trow = lax.broadcasted_iota(jnp.int32, (1, 128), 1).astype(jnp.float32) * float(_TM)
        act_ref[...] = (trow < total).astype(jnp.int32)

    e = e_ref[...]
    lane = lax.broadcasted_iota(jnp.int32, (_TM, _E), 1).astype(jnp.float32)
    oh = (lane == e).astype(jnp.float32)
    off_tok = jnp.sum(oh * offs_s[...], axis=-1, keepdims=True)
    p_ref[...] = (off_tok + rank_ref[...]).astype(jnp.int32)


def _ffn_body(te_ref, act_ref, xs_ref, w1_ref, w2_ref, o_ref, w1c, w2c, eid_s):
    t = pl.program_id(0)

    @pl.when(t == 0)
    def _():
        eid_s[0] = -1

    e = te_ref[t]
    active = act_ref[t] == 1

    @pl.when(jnp.logical_and(active, eid_s[0] != e))
    def _():
        w1c[...] = w1_ref[0].astype(jnp.bfloat16)
        w2c[...] = w2_ref[0].astype(jnp.bfloat16)
        eid_s[0] = e

    @pl.when(active)
    def _():
        xb16 = xs_ref[...].astype(jnp.bfloat16)
        h = jnp.dot(xb16, w1c[...], preferred_element_type=jnp.float32)
        h16 = jnp.maximum(h, 0.0).astype(jnp.bfloat16)
        o_ref[...] = jnp.dot(h16, w2c[...], preferred_element_type=jnp.float32)


@functools.cache
def _sc_kernels():
    mesh = plsc.VectorSubcoreMesh(core_axis_name="c", subcore_axis_name="s")
    scratch = [
        pltpu.VMEM((_NCH, _CH), jnp.int32),
        pltpu.VMEM((_CH, _D), jnp.float32),
        pltpu.SemaphoreType.DMA,
    ]

    @functools.partial(
        pl.kernel,
        out_type=jax.ShapeDtypeStruct((_NPAD, _D), jnp.float32),
        mesh=mesh,
        scratch_types=scratch,
    )
    def sc_scatter(xg_hbm, p_hbm, xs_hbm, idx_v, rows_v, sem):
        wid = lax.axis_index("s") * 2 + lax.axis_index("c")
        base = wid * _RPW
        pltpu.sync_copy(p_hbm.at[wid], idx_v)
        for c in range(_NCH):
            pltpu.sync_copy(xg_hbm.at[pl.ds(base + c * _CH, _CH)], rows_v)
            pltpu.async_copy(rows_v, xs_hbm.at[idx_v.at[c]], sem).wait()

    @functools.partial(
        pl.kernel,
        out_type=jax.ShapeDtypeStruct((_N, _D), jnp.float32),
        mesh=mesh,
        scratch_types=scratch,
    )
    def sc_gather(os_hbm, p_hbm, y_hbm, idx_v, rows_v, sem):
        wid = lax.axis_index("s") * 2 + lax.axis_index("c")
        base = wid * _RPW
        pltpu.sync_copy(p_hbm.at[wid], idx_v)
        for c in range(_NCH):
            pltpu.async_copy(os_hbm.at[idx_v.at[c]], rows_v, sem).wait()
            pltpu.sync_copy(rows_v, y_hbm.at[pl.ds(base + c * _CH, _CH)])

    return sc_scatter, sc_gather


def kernel(x, token_mask, router_w, w1, w2):
    Bc, Tc, Dc = x.shape
    xf = x.reshape(_N, _D)
    maskf = token_mask.reshape(_N, 1).astype(jnp.float32)

    router_call = pl.pallas_call(
        _router_body,
        grid=(_NB,),
        in_specs=[
            pl.BlockSpec((_TM, _D), lambda i: (i, 0)),
            pl.BlockSpec((_E, _D), lambda i: (0, 0)),
            pl.BlockSpec((_TM, 1), lambda i: (i, 0)),
        ],
        out_specs=[
            pl.BlockSpec((_TM, _D), lambda i: (i, 0)),
            pl.BlockSpec((_TM, 1), lambda i: (i, 0)),
            pl.BlockSpec((_TM, 1), lambda i: (i, 0)),
            pl.BlockSpec((1, _E), lambda i: (0, 0)),
        ],
        out_shape=[
            jax.ShapeDtypeStruct((_N, _D), jnp.float32),
            jax.ShapeDtypeStruct((_N, 1), jnp.float32),
            jax.ShapeDtypeStruct((_N, 1), jnp.float32),
            jax.ShapeDtypeStruct((1, _E), jnp.float32),
        ],
        scratch_shapes=[pltpu.VMEM((1, _E), jnp.float32)],
    )
    xg, ef, rank, counts = router_call(xf, router_w, maskf)

    plan_call = pl.pallas_call(
        _plan_body,
        grid=(_NB,),
        in_specs=[
            pl.BlockSpec((1, _E), lambda i: (0, 0)),
            pl.BlockSpec((_TM, 1), lambda i: (i, 0)),
            pl.BlockSpec((_TM, 1), lambda i: (i, 0)),
        ],
        out_specs=[
            pl.BlockSpec((_TM, 1), lambda i: (i, 0)),
            pl.BlockSpec((1, 128), lambda i: (0, 0)),
            pl.BlockSpec((1, 128), lambda i: (0, 0)),
        ],
        out_shape=[
            jax.ShapeDtypeStruct((_N, 1), jnp.int32),
            jax.ShapeDtypeStruct((1, 128), jnp.int32),
            jax.ShapeDtypeStruct((1, 128), jnp.int32),
        ],
        scratch_shapes=[pltpu.VMEM((1, _E), jnp.float32)],
    )
    p, te_pad, act_pad = plan_call(counts, ef, rank)

    p3 = p.reshape(_NW, _NCH, _CH)
    te = te_pad.reshape(128)[:_NT]
    act = act_pad.reshape(128)[:_NT]

    sc_scatter, sc_gather = _sc_kernels()
    xs = sc_scatter(xg, p3)

    ffn_call = pl.pallas_call(
        _ffn_body,
        grid_spec=pltpu.PrefetchScalarGridSpec(
            num_scalar_prefetch=2,
            grid=(_NT,),
            in_specs=[
                pl.BlockSpec((_TM, _D), lambda t, te_r, act_r: (t, 0)),
                pl.BlockSpec((1, _D, _F), lambda t, te_r, act_r: (te_r[t], 0, 0)),
                pl.BlockSpec((1, _F, _D), lambda t, te_r, act_r: (te_r[t], 0, 0)),
            ],
            out_specs=pl.BlockSpec((_TM, _D), lambda t, te_r, act_r: (t, 0)),
            scratch_shapes=[
                pltpu.VMEM((_D, _F), jnp.bfloat16),
                pltpu.VMEM((_F, _D), jnp.bfloat16),
                pltpu.SMEM((1,), jnp.int32),
            ],
        ),
        out_shape=jax.ShapeDtypeStruct((_NPAD, _D), jnp.float32),
        compiler_params=pltpu.CompilerParams(
            dimension_semantics=("arbitrary",)),
    )
    osrt = ffn_call(te, act, xs, w1, w2)

    y = sc_gather(osrt, p3)
    return y.reshape(Bc, Tc, Dc)
